# baseline (speedup 1.0000x reference)
import functools
import jax
import jax.numpy as jnp
from jax import lax
from jax.experimental import pallas as pl
from jax.experimental.pallas import tpu as pltpu
from jax.experimental.pallas import tpu_sc as plsc

R = 32
LANES = 16


def _sc_body(x_hbm, pe_hbm, o_hbm, xbuf, pebuf, *, rows_pw, seq_len, dim):
    wid = lax.axis_index("s") * 2 + lax.axis_index("c")
    base = wid * rows_pw
    nchunks = rows_pw // R

    def chunk(g, _):
        row0 = base + g * R
        pe0 = lax.rem(row0, seq_len)
        pltpu.sync_copy(x_hbm.at[pl.ds(row0, R), :], xbuf)
        pltpu.sync_copy(pe_hbm.at[pl.ds(pe0, R), :], pebuf)

        def row_add(r, _):
            for c in range(dim // LANES):
                sl = pl.ds(c * LANES, LANES)
                xbuf[r, sl] = xbuf[r, sl] + pebuf[r, sl]
            return 0

        lax.fori_loop(0, R, row_add, 0)
        pltpu.sync_copy(xbuf, o_hbm.at[pl.ds(row0, R), :])
        return 0

    lax.fori_loop(0, nchunks, chunk, 0)


def kernel(x, pe_table, position_ids):
    del position_ids
    batch, seq_len, dim = x.shape
    rows = batch * seq_len
    info = plsc.get_sparse_core_info()
    nw = info.num_cores * info.num_subcores
    rows_pw = rows // nw
    body = functools.partial(
        _sc_body, rows_pw=rows_pw, seq_len=seq_len, dim=dim)
    fn = pl.kernel(
        body,
        out_type=jax.ShapeDtypeStruct((rows, dim), x.dtype),
        mesh=plsc.VectorSubcoreMesh(core_axis_name="c", subcore_axis_name="s"),
        scratch_types=[
            pltpu.VMEM((R, dim), x.dtype),
            pltpu.VMEM((R, dim), x.dtype),
        ],
    )
    out = fn(x.reshape(rows, dim), pe_table[:seq_len])
    return out.reshape(batch, seq_len, dim)
